# baseline (speedup 1.0000x reference)
import jax
import jax.numpy as jnp
from jax.experimental import pallas as pl
from jax.experimental.pallas import tpu as pltpu


_BM = 400
_NPANEL = 24
_CR = 80
_NCHUNK = 5


def _gcn_kernel(adj_ref, x_hbm_ref, w_hbm_ref, b_hbm_ref, out_hbm_ref,
                h_ref, x_ref, w_ref, b_ref, buf_ref, tail_ref, obuf_ref,
                otail_ref, sem_xwb_ref, sem_adj_ref, sem_tail_ref,
                sem_out_ref, sem_otail_ref):
    base = _NPANEL * _BM

    def panel_copy(p):
        return pltpu.make_async_copy(
            adj_ref.at[pl.ds(p * _BM, _BM), :],
            buf_ref.at[p % 2],
            sem_adj_ref.at[p % 2],
        )

    def chunk_copy(k):
        return pltpu.make_async_copy(
            adj_ref.at[pl.ds(base + k * _CR, _CR), :],
            tail_ref.at[k],
            sem_tail_ref.at[k],
        )

    def out_copy(p):
        return pltpu.make_async_copy(
            obuf_ref.at[p % 2],
            out_hbm_ref.at[pl.ds(p * _BM, _BM), :],
            sem_out_ref.at[p % 2],
        )

    cx = pltpu.make_async_copy(x_hbm_ref, x_ref, sem_xwb_ref.at[0])
    cw = pltpu.make_async_copy(w_hbm_ref, w_ref, sem_xwb_ref.at[1])
    cb = pltpu.make_async_copy(b_hbm_ref, b_ref, sem_xwb_ref.at[2])
    cx.start()
    cw.start()
    cb.start()
    panel_copy(0).start()
    panel_copy(1).start()

    cx.wait()
    cw.wait()
    cb.wait()
    h_ref[...] = jnp.dot(
        x_ref[...], w_ref[...], preferred_element_type=jnp.float32
    )

    for p in range(_NPANEL):
        panel_copy(p).wait()
        if p >= 2:
            out_copy(p - 2).wait()
        obuf_ref[p % 2] = (
            jnp.dot(buf_ref[p % 2], h_ref[...],
                    preferred_element_type=jnp.float32)
            + b_ref[...]
        )
        out_copy(p).start()
        if p + 2 < _NPANEL:
            panel_copy(p + 2).start()
        elif p + 2 == _NPANEL:
            for k in range(_NCHUNK):
                chunk_copy(k).start()

    for k in range(_NCHUNK):
        chunk_copy(k).wait()
        otail_ref[pl.ds(k * _CR, _CR), :] = (
            jnp.dot(tail_ref[k], h_ref[...],
                    preferred_element_type=jnp.float32)
            + b_ref[...]
        )
    ct = pltpu.make_async_copy(
        otail_ref, out_hbm_ref.at[pl.ds(base, _NCHUNK * _CR), :],
        sem_otail_ref,
    )
    ct.start()

    out_copy(_NPANEL - 2).wait()
    out_copy(_NPANEL - 1).wait()
    ct.wait()


def kernel(x, adj, W, b):
    n, d_in = x.shape
    d_out = W.shape[1]
    out = pl.pallas_call(
        _gcn_kernel,
        in_specs=[
            pl.BlockSpec(memory_space=pltpu.MemorySpace.HBM),
            pl.BlockSpec(memory_space=pltpu.MemorySpace.HBM),
            pl.BlockSpec(memory_space=pltpu.MemorySpace.HBM),
            pl.BlockSpec(memory_space=pltpu.MemorySpace.HBM),
        ],
        out_specs=pl.BlockSpec(memory_space=pltpu.MemorySpace.HBM),
        out_shape=jax.ShapeDtypeStruct((n, d_out), jnp.float32),
        scratch_shapes=[
            pltpu.VMEM((n, d_out), jnp.float32),
            pltpu.VMEM((n, d_in), jnp.float32),
            pltpu.VMEM((d_in, d_out), jnp.float32),
            pltpu.VMEM((1, d_out), jnp.float32),
            pltpu.VMEM((2, _BM, n), jnp.float32),
            pltpu.VMEM((_NCHUNK, _CR, n), jnp.float32),
            pltpu.VMEM((2, _BM, d_out), jnp.float32),
            pltpu.VMEM((_NCHUNK * _CR, d_out), jnp.float32),
            pltpu.SemaphoreType.DMA((3,)),
            pltpu.SemaphoreType.DMA((2,)),
            pltpu.SemaphoreType.DMA((_NCHUNK,)),
            pltpu.SemaphoreType.DMA((2,)),
            pltpu.SemaphoreType.DMA,
        ],
        compiler_params=pltpu.CompilerParams(
            vmem_limit_bytes=64 * 1024 * 1024,
        ),
    )(adj, x, W, b.reshape(1, d_out))
    return out.reshape(1, n, d_out)


# speedup vs baseline: 1.0059x; 1.0059x over previous
import jax
import jax.numpy as jnp
from jax.experimental import pallas as pl
from jax.experimental.pallas import tpu as pltpu


_BM = 400
_NPANEL = 24
_CR = 80
_NCHUNK = 5
_NSLOT = 2


def _gcn_kernel(adj_win_ref, adj_hbm_ref, x_hbm_ref, w_hbm_ref, b_hbm_ref,
                out_ref, h_ref, x_ref, w_ref, b_ref, tail_ref,
                sem_xwb_ref, sem_tail_ref):
    i = pl.program_id(0)
    base = _NPANEL * _BM
    nk = h_ref.shape[0]
    half = nk // 2

    @pl.when(i == 0)
    def _load_h_and_panel0():
        cw = pltpu.make_async_copy(w_hbm_ref, w_ref, sem_xwb_ref.at[0])
        cx0 = pltpu.make_async_copy(
            x_hbm_ref.at[pl.ds(0, half), :],
            x_ref.at[pl.ds(0, half), :], sem_xwb_ref.at[1])
        cx1 = pltpu.make_async_copy(
            x_hbm_ref.at[pl.ds(half, half), :],
            x_ref.at[pl.ds(half, half), :], sem_xwb_ref.at[2])
        cb = pltpu.make_async_copy(b_hbm_ref, b_ref, sem_xwb_ref.at[3])
        cw.start()
        cx0.start()
        cx1.start()
        cb.start()
        cw.wait()
        cx0.wait()
        h_ref[pl.ds(0, half), :] = jnp.dot(
            x_ref[pl.ds(0, half), :], w_ref[...],
            preferred_element_type=jnp.float32)
        acc0 = jnp.dot(adj_win_ref[:, pl.ds(0, half)],
                       h_ref[pl.ds(0, half), :],
                       preferred_element_type=jnp.float32)
        cx1.wait()
        h_ref[pl.ds(half, half), :] = jnp.dot(
            x_ref[pl.ds(half, half), :], w_ref[...],
            preferred_element_type=jnp.float32)
        acc1 = jnp.dot(adj_win_ref[:, pl.ds(half, half)],
                       h_ref[pl.ds(half, half), :],
                       preferred_element_type=jnp.float32)
        cb.wait()
        out_ref[...] = acc0 + acc1 + b_ref[...]

    @pl.when(i == _NPANEL - 1)
    def _issue_tail():
        for s in range(_NSLOT):
            pltpu.make_async_copy(
                adj_hbm_ref.at[pl.ds(base + s * _CR, _CR), :],
                tail_ref.at[s],
                sem_tail_ref.at[s],
            ).start()

    @pl.when(jnp.logical_and(i >= 1, i < _NPANEL))
    def _main():
        out_ref[...] = (
            jnp.dot(adj_win_ref[...], h_ref[...],
                    preferred_element_type=jnp.float32)
            + b_ref[...]
        )

    @pl.when(i == _NPANEL)
    def _tail():
        def body(k, carry):
            slot = jax.lax.rem(k, _NSLOT)
            pltpu.make_async_copy(
                adj_hbm_ref.at[pl.ds(base + k * _CR, _CR), :],
                tail_ref.at[slot],
                sem_tail_ref.at[slot],
            ).wait()
            out_ref[pl.ds(k * _CR, _CR), :] = (
                jnp.dot(tail_ref[slot], h_ref[...],
                        preferred_element_type=jnp.float32)
                + b_ref[...]
            )

            @pl.when(k + _NSLOT < _NCHUNK)
            def _issue_next():
                pltpu.make_async_copy(
                    adj_hbm_ref.at[pl.ds(base + (k + _NSLOT) * _CR, _CR), :],
                    tail_ref.at[slot],
                    sem_tail_ref.at[slot],
                ).start()

            return carry

        jax.lax.fori_loop(0, _NCHUNK, body, 0)


def kernel(x, adj, W, b):
    n, d_in = x.shape
    d_out = W.shape[1]
    out = pl.pallas_call(
        _gcn_kernel,
        grid=(_NPANEL + 1,),
        in_specs=[
            pl.BlockSpec((_BM, n), lambda i: (jnp.minimum(i, _NPANEL - 1), 0)),
            pl.BlockSpec(memory_space=pltpu.MemorySpace.HBM),
            pl.BlockSpec(memory_space=pltpu.MemorySpace.HBM),
            pl.BlockSpec(memory_space=pltpu.MemorySpace.HBM),
            pl.BlockSpec(memory_space=pltpu.MemorySpace.HBM),
        ],
        out_specs=pl.BlockSpec((_BM, d_out), lambda i: (i, 0)),
        out_shape=jax.ShapeDtypeStruct((n, d_out), jnp.float32),
        scratch_shapes=[
            pltpu.VMEM((n, d_out), jnp.float32),
            pltpu.VMEM((n, d_in), jnp.float32),
            pltpu.VMEM((d_in, d_out), jnp.float32),
            pltpu.VMEM((1, d_out), jnp.float32),
            pltpu.VMEM((_NSLOT, _CR, n), jnp.float32),
            pltpu.SemaphoreType.DMA((4,)),
            pltpu.SemaphoreType.DMA((_NSLOT,)),
        ],
        compiler_params=pltpu.CompilerParams(
            vmem_limit_bytes=64 * 1024 * 1024,
        ),
    )(adj, adj, x, W, b.reshape(1, d_out))
    return out.reshape(1, n, d_out)


# speedup vs baseline: 1.0477x; 1.0415x over previous
import jax
import jax.numpy as jnp
from jax.experimental import pallas as pl
from jax.experimental.pallas import tpu as pltpu


_BM = 400
_NPANEL = 24
_CR = 80
_NCHUNK = 5
_NSLOT = 3
_WARM = 2


def _gcn_kernel(adj_win_ref, x_ref, w_ref, b_ref, adj_hbm_ref, out_ref,
                h_ref, tail_ref, sem_tail_ref):
    i = pl.program_id(0)
    base = _NPANEL * _BM
    half = h_ref.shape[0] // 2

    @pl.when(i == 0)
    def _h_first_half():
        h_ref[pl.ds(0, half), :] = jnp.dot(
            x_ref[0], w_ref[...], preferred_element_type=jnp.float32)

    @pl.when(i == 1)
    def _h_second_half():
        h_ref[pl.ds(half, half), :] = jnp.dot(
            x_ref[0], w_ref[...], preferred_element_type=jnp.float32)

    @pl.when(i == _WARM + _NPANEL - 1)
    def _issue_tail():
        for s in range(_NSLOT):
            pltpu.make_async_copy(
                adj_hbm_ref.at[pl.ds(base + s * _CR, _CR), :],
                tail_ref.at[s],
                sem_tail_ref.at[s],
            ).start()

    @pl.when(jnp.logical_and(i >= _WARM, i < _WARM + _NPANEL))
    def _main():
        out_ref[...] = (
            jnp.dot(adj_win_ref[...], h_ref[...],
                    preferred_element_type=jnp.float32)
            + b_ref[...]
        )

    @pl.when(i == _WARM + _NPANEL)
    def _tail():
        def body(k, carry):
            slot = jax.lax.rem(k, _NSLOT)
            pltpu.make_async_copy(
                adj_hbm_ref.at[pl.ds(base + k * _CR, _CR), :],
                tail_ref.at[slot],
                sem_tail_ref.at[slot],
            ).wait()
            out_ref[pl.ds(k * _CR, _CR), :] = (
                jnp.dot(tail_ref[slot], h_ref[...],
                        preferred_element_type=jnp.float32)
                + b_ref[...]
            )

            @pl.when(k + _NSLOT < _NCHUNK)
            def _issue_next():
                pltpu.make_async_copy(
                    adj_hbm_ref.at[pl.ds(base + (k + _NSLOT) * _CR, _CR), :],
                    tail_ref.at[slot],
                    sem_tail_ref.at[slot],
                ).start()

            return carry

        jax.lax.fori_loop(0, _NCHUNK, body, 0)


def kernel(x, adj, W, b):
    n, d_in = x.shape
    d_out = W.shape[1]
    half = n // 2
    npan = _NPANEL
    out = pl.pallas_call(
        _gcn_kernel,
        grid=(_WARM + _NPANEL + 1,),
        in_specs=[
            pl.BlockSpec((_BM, n),
                         lambda i: (jnp.clip(i - _WARM, 0, npan - 1), 0)),
            pl.BlockSpec((1, half, d_in),
                         lambda i: (jnp.minimum(i, 1), 0, 0)),
            pl.BlockSpec((d_in, d_out), lambda i: (0, 0)),
            pl.BlockSpec((1, d_out), lambda i: (0, 0)),
            pl.BlockSpec(memory_space=pltpu.MemorySpace.HBM),
        ],
        out_specs=pl.BlockSpec(
            (_BM, d_out),
            lambda i: (jnp.clip(i - _WARM, 0, n // _BM - 1), 0)),
        out_shape=jax.ShapeDtypeStruct((n, d_out), jnp.float32),
        scratch_shapes=[
            pltpu.VMEM((n, d_out), jnp.float32),
            pltpu.VMEM((_NSLOT, _CR, n), jnp.float32),
            pltpu.SemaphoreType.DMA((_NSLOT,)),
        ],
        compiler_params=pltpu.CompilerParams(
            vmem_limit_bytes=64 * 1024 * 1024,
        ),
    )(adj, x.reshape(2, half, d_in), W, b.reshape(1, d_out), adj)
    return out.reshape(1, n, d_out)


# speedup vs baseline: 1.0663x; 1.0178x over previous
import jax
import jax.numpy as jnp
from jax.experimental import pallas as pl
from jax.experimental.pallas import tpu as pltpu


_BM = 400
_NPANEL = 24
_CR = 80
_NCHUNK = 5
_NSLOT = 3


def _gcn_kernel(x_ref, w_ref, b_ref, adj_win_ref, adj_hbm_ref, out_ref,
                h_ref, tail_ref, sem_ref):
    i = pl.program_id(0)
    base = _NPANEL * _BM

    @pl.when(i == 0)
    def _compute_h():
        h_ref[...] = jnp.dot(
            x_ref[...], w_ref[...], preferred_element_type=jnp.float32
        )

    @pl.when(i == _NPANEL - 1)
    def _issue_tail():
        for s in range(_NSLOT):
            pltpu.make_async_copy(
                adj_hbm_ref.at[pl.ds(base + s * _CR, _CR), :],
                tail_ref.at[s],
                sem_ref.at[s],
            ).start()

    @pl.when(i < _NPANEL)
    def _main():
        out_ref[...] = (
            jnp.dot(adj_win_ref[...], h_ref[...],
                    preferred_element_type=jnp.float32)
            + b_ref[...]
        )

    @pl.when(i == _NPANEL)
    def _tail():
        def body(k, carry):
            slot = jax.lax.rem(k, _NSLOT)
            pltpu.make_async_copy(
                adj_hbm_ref.at[pl.ds(base + k * _CR, _CR), :],
                tail_ref.at[slot],
                sem_ref.at[slot],
            ).wait()
            out_ref[pl.ds(k * _CR, _CR), :] = (
                jnp.dot(tail_ref[slot], h_ref[...],
                        preferred_element_type=jnp.float32)
                + b_ref[...]
            )

            @pl.when(k + _NSLOT < _NCHUNK)
            def _issue_next():
                pltpu.make_async_copy(
                    adj_hbm_ref.at[pl.ds(base + (k + _NSLOT) * _CR, _CR), :],
                    tail_ref.at[slot],
                    sem_ref.at[slot],
                ).start()

            return carry

        jax.lax.fori_loop(0, _NCHUNK, body, 0)


def kernel(x, adj, W, b):
    n, d_in = x.shape
    d_out = W.shape[1]
    out = pl.pallas_call(
        _gcn_kernel,
        grid=(_NPANEL + 1,),
        in_specs=[
            pl.BlockSpec((n, d_in), lambda i: (0, 0)),
            pl.BlockSpec((d_in, d_out), lambda i: (0, 0)),
            pl.BlockSpec((1, d_out), lambda i: (0, 0)),
            pl.BlockSpec((_BM, n), lambda i: (jnp.minimum(i, _NPANEL - 1), 0)),
            pl.BlockSpec(memory_space=pltpu.MemorySpace.HBM),
        ],
        out_specs=pl.BlockSpec((_BM, d_out), lambda i: (i, 0)),
        out_shape=jax.ShapeDtypeStruct((n, d_out), jnp.float32),
        scratch_shapes=[
            pltpu.VMEM((n, d_out), jnp.float32),
            pltpu.VMEM((_NSLOT, _CR, n), jnp.float32),
            pltpu.SemaphoreType.DMA((_NSLOT,)),
        ],
        compiler_params=pltpu.CompilerParams(
            vmem_limit_bytes=64 * 1024 * 1024,
        ),
    )(x, W, b.reshape(1, d_out), adj, adj)
    return out.reshape(1, n, d_out)
